# baseline (speedup 1.0000x reference)
import functools
import math

import jax
import jax.numpy as jnp
from jax import lax
from jax.experimental import pallas as pl
from jax.experimental.pallas import tpu as pltpu
from jax.experimental.pallas import tpu_sc as plsc

B = 8
N = 197
D = 192
NH = 3
DH = 64
L = 6
NE = 16
HD = 768
CAP = 197
CAPP = 256
NP = 256
NTOK = B * NP
NW = 32
PERW = NTOK // NW

_f32 = jnp.float32


def _ln(x, g, b):
    m = x.mean(-1, keepdims=True)
    v = ((x - m) ** 2).mean(-1, keepdims=True)
    return (x - m) / jnp.sqrt(v + 1e-6) * g + b


def _embed_body(xp_ref, wp_ref, bp_ref, cls_ref, pos_ref, out_ref):
    y = jnp.dot(xp_ref[...], wp_ref[...]) + bp_ref[...]
    for b in range(B):
        out_ref[b, 0:1, :] = cls_ref[...] + pos_ref[0:1, :]
        out_ref[b, 1:N, :] = y[b * (N - 1):(b + 1) * (N - 1), :] + pos_ref[1:N, :]


def _embed(xp, wp, bp, cls, pos):
    return pl.pallas_call(
        _embed_body,
        out_shape=jax.ShapeDtypeStruct((B, N, D), _f32),
    )(xp, wp, bp, cls, pos)


def _layer_body(t_ref, g_ref, scl_ref, ln1g_ref, ln1b_ref, wqkv_ref, bqkv_ref,
                wproj_ref, bproj_ref, ln2g_ref, ln2b_ref, wg_ref,
                tmid_ref, u_ref, loc_ref, sclo_ref, aux_ref):
    iota_e = lax.broadcasted_iota(jnp.int32, (N, NE), 1).astype(_f32)
    ii = lax.broadcasted_iota(jnp.int32, (N, N), 0)
    jj = lax.broadcasted_iota(jnp.int32, (N, N), 1)
    tril = (jj < ii).astype(_f32)
    lane = lax.broadcasted_iota(jnp.int32, (NP,), 0)

    off = jnp.zeros((NE,), _f32)
    imp = jnp.zeros((NE,), _f32)
    for b in range(B):
        sb = scl_ref[b][:N][:, None]
        tb = t_ref[b] + jnp.where(sb > 0.0, g_ref[b, :N, :] * sb, 0.0)
        u1 = _ln(tb, ln1g_ref[...], ln1b_ref[...])
        qkv = jnp.dot(u1, wqkv_ref[...]) + bqkv_ref[...]
        outs = []
        for h in range(NH):
            q = qkv[:, h * DH:(h + 1) * DH]
            k = qkv[:, D + h * DH:D + (h + 1) * DH]
            v = qkv[:, 2 * D + h * DH:2 * D + (h + 1) * DH]
            s = jnp.dot(q, k.T) * (1.0 / math.sqrt(DH))
            p = jax.nn.softmax(s, axis=-1)
            outs.append(jnp.dot(p, v))
        o = jnp.concatenate(outs, axis=1)
        tm = tb + jnp.dot(o, wproj_ref[...]) + bproj_ref[...]
        tmid_ref[b] = tm
        u2 = _ln(tm, ln2g_ref[...], ln2b_ref[...])
        u_ref[b, :N, :] = u2

        logits = jnp.dot(u2, wg_ref[...])
        probs = jax.nn.softmax(logits, axis=-1)
        gate = jnp.max(probs, axis=-1)
        eq = probs == gate[:, None]
        idxf = jnp.min(jnp.where(eq, iota_e, 1e9), axis=-1)
        oh = (iota_e == idxf[:, None]).astype(_f32)
        cnt = jnp.dot(tril, oh)
        pos = jnp.sum((cnt + off[None, :]) * oh, axis=-1)
        keep = (pos < float(CAP)).astype(_f32)
        posc = jnp.minimum(pos, float(CAPP - 1))
        locb = (idxf * CAPP + posc).astype(jnp.int32)
        sclb = gate * keep
        loc_full = jnp.concatenate(
            [locb, jnp.full((NP - N,), CAPP - 1, jnp.int32)])
        scl_full = jnp.concatenate([sclb, jnp.zeros((NP - N,), _f32)])
        loc_ref[b] = jnp.where(lane < N, loc_full, CAPP - 1)
        sclo_ref[b] = jnp.where(lane < N, scl_full, 0.0)
        off = off + jnp.sum(oh, axis=0)
        imp = imp + jnp.sum(probs, axis=0)

    tot = float(B * N)
    aux = float(NE) * jnp.sum((imp / tot) * (off / tot))
    aux_ref[...] = aux.reshape(1, 1)


def _layer_head(t, g, scl, ln1g, ln1b, wqkv, bqkv, wproj, bproj, ln2g, ln2b, wg):
    return pl.pallas_call(
        _layer_body,
        out_shape=(
            jax.ShapeDtypeStruct((B, N, D), _f32),
            jax.ShapeDtypeStruct((B, NP, D), _f32),
            jax.ShapeDtypeStruct((B, NP), jnp.int32),
            jax.ShapeDtypeStruct((B, NP), _f32),
            jax.ShapeDtypeStruct((1, 1), _f32),
        ),
    )(t, g, scl, ln1g, ln1b, wqkv, bqkv, wproj, bproj, ln2g, ln2b, wg)


def _ffn_body(buf_ref, w1_ref, b1_ref, w2_ref, b2_ref, out_ref):
    x = buf_ref[...]
    h = jax.nn.gelu(jnp.dot(x, w1_ref[0]) + b1_ref[0])
    out_ref[...] = jnp.dot(h, w2_ref[0]) + b2_ref[0]


def _ffn(buf, w1, b1, w2, b2):
    return pl.pallas_call(
        _ffn_body,
        grid=(NE,),
        in_specs=[
            pl.BlockSpec((CAPP, D), lambda e: (e, 0)),
            pl.BlockSpec((1, D, HD), lambda e: (e, 0, 0)),
            pl.BlockSpec((1, 1, HD), lambda e: (e, 0, 0)),
            pl.BlockSpec((1, HD, D), lambda e: (e, 0, 0)),
            pl.BlockSpec((1, 1, D), lambda e: (e, 0, 0)),
        ],
        out_specs=pl.BlockSpec((CAPP, D), lambda e: (e, 0)),
        out_shape=jax.ShapeDtypeStruct((NE * CAPP, D), _f32),
    )(buf, w1, b1.reshape(NE, 1, HD), w2, b2.reshape(NE, 1, D))


def _head(t, g, scl, lnfg, lnfb, wh, bh, auxv):
    def body(t_ref, g_ref, scl_ref, lnfg_ref, lnfb_ref, wh_ref, bh_ref,
             aux_ref, logits_ref, cv_ref):
        rows = []
        for b in range(B):
            sb = scl_ref[b, 0]
            tb = t_ref[b, 0:1, :] + jnp.where(sb > 0.0,
                                              g_ref[b, 0:1, :] * sb, 0.0)
            rows.append(tb)
        tc = jnp.concatenate(rows, axis=0)
        tc = _ln(tc, lnfg_ref[...], lnfb_ref[...])
        logits_ref[...] = jnp.dot(tc, wh_ref[...]) + bh_ref[...]
        cv_ref[...] = jnp.sum(aux_ref[...]).reshape(1, 1)

    return pl.pallas_call(
        body,
        out_shape=(
            jax.ShapeDtypeStruct((B, 1000), _f32),
            jax.ShapeDtypeStruct((1, 1), _f32),
        ),
    )(t, g, scl, lnfg, lnfb, wh, bh, auxv)


@functools.lru_cache(maxsize=1)
def _sc_kernels():
    mesh = plsc.VectorSubcoreMesh(core_axis_name="c", subcore_axis_name="s")
    cparams = pltpu.CompilerParams(use_tc_tiling_on_sc=False)
    scratch = [
        pltpu.VMEM((PERW,), jnp.int32),
        pltpu.VMEM((PERW, D), _f32),
        pltpu.SemaphoreType.DMA,
    ]

    @functools.partial(
        pl.kernel,
        out_type=jax.ShapeDtypeStruct((NE * CAPP, D), _f32),
        mesh=mesh, scratch_types=scratch, compiler_params=cparams)
    def dispatch(u_hbm, loc_hbm, buf_hbm, loc_v, u_v, sem):
        wid = lax.axis_index("s") * 2 + lax.axis_index("c")
        base = wid * PERW
        pltpu.sync_copy(loc_hbm.at[pl.ds(base, PERW)], loc_v)
        pltpu.sync_copy(u_hbm.at[pl.ds(base, PERW)], u_v)
        pltpu.async_copy(u_v, buf_hbm.at[loc_v], sem).wait()

    @functools.partial(
        pl.kernel,
        out_type=jax.ShapeDtypeStruct((NTOK, D), _f32),
        mesh=mesh, scratch_types=scratch, compiler_params=cparams)
    def combine(outbuf_hbm, loc_hbm, g_hbm, loc_v, rows_v, sem):
        wid = lax.axis_index("s") * 2 + lax.axis_index("c")
        base = wid * PERW
        pltpu.sync_copy(loc_hbm.at[pl.ds(base, PERW)], loc_v)
        pltpu.async_copy(outbuf_hbm.at[loc_v], rows_v, sem).wait()
        pltpu.sync_copy(rows_v, g_hbm.at[pl.ds(base, PERW)])

    return dispatch, combine


def kernel(x, params):
    p = params
    gh = 224 // 16
    xp = (x.reshape(B, 3, gh, 16, gh, 16)
           .transpose(0, 2, 4, 1, 3, 5)
           .reshape(B * gh * gh, 3 * 16 * 16))
    t = _embed(xp, p['Wpatch'], p['bpatch'], p['cls'].reshape(1, D),
               p['pos'].reshape(N, D))

    g = jnp.zeros((B, NP, D), _f32)
    scl = jnp.zeros((B, NP), _f32)
    auxs = []
    for l in range(L):
        t, u, loc2d, scl2d, aux_l = _layer_head(
            t, g, scl, p['ln1_g'][l], p['ln1_b'][l], p['Wqkv'][l],
            p['bqkv'][l], p['Wproj'][l], p['bproj'][l], p['ln2_g'][l],
            p['ln2_b'][l], p['Wg'][l])
        dispatch, combine = _sc_kernels()
        loc = loc2d.reshape(NTOK)
        buf = dispatch(u.reshape(NTOK, D), loc)
        outbuf = _ffn(buf, p['W1'][l], p['b1'][l], p['W2'][l], p['b2'][l])
        g = combine(outbuf, loc).reshape(B, NP, D)
        scl = scl2d
        auxs.append(aux_l.reshape(1))
    auxv = jnp.concatenate(auxs + [jnp.zeros((2,), _f32)]).reshape(1, 8)
    logits, cv = _head(t, g, scl, p['lnf_g'], p['lnf_b'], p['Whead'],
                       p['bhead'], auxv)
    return logits, cv.reshape(())


# speedup vs baseline: 1.1638x; 1.1638x over previous
import functools
import math

import jax
import jax.numpy as jnp
from jax import lax
from jax.experimental import pallas as pl
from jax.experimental.pallas import tpu as pltpu
from jax.experimental.pallas import tpu_sc as plsc

B = 8
N = 197
D = 192
NH = 3
DH = 64
L = 6
NE = 16
HD = 768
CAP = 197
CAPP = 256
NP = 256
NTOK = B * NP
NW = 32
PERW = NTOK // NW

_f32 = jnp.float32


def _ln(x, g, b):
    m = x.mean(-1, keepdims=True)
    v = ((x - m) ** 2).mean(-1, keepdims=True)
    return (x - m) / jnp.sqrt(v + 1e-6) * g + b


def _embed_body(xp_ref, wp_ref, bp_ref, cls_ref, pos_ref, out_ref):
    y = jnp.dot(xp_ref[...], wp_ref[...]) + bp_ref[...]
    for b in range(B):
        out_ref[b, 0:1, :] = cls_ref[...] + pos_ref[0:1, :]
        out_ref[b, 1:N, :] = y[b * (N - 1):(b + 1) * (N - 1), :] + pos_ref[1:N, :]


def _embed(xp, wp, bp, cls, pos):
    return pl.pallas_call(
        _embed_body,
        out_shape=jax.ShapeDtypeStruct((B, N, D), _f32),
    )(xp, wp, bp, cls, pos)


def _layer_body(t_ref, g_ref, scl_ref, ln1g_ref, ln1b_ref, wqkv_ref, bqkv_ref,
                wproj_ref, bproj_ref, ln2g_ref, ln2b_ref, wg_ref,
                tmid_ref, u_ref, loc_ref, sclo_ref, aux_ref):
    iota_e = lax.broadcasted_iota(jnp.int32, (N, NE), 1).astype(_f32)
    ii = lax.broadcasted_iota(jnp.int32, (N, N), 0)
    jj = lax.broadcasted_iota(jnp.int32, (N, N), 1)
    tril = (jj < ii).astype(_f32)
    lane = lax.broadcasted_iota(jnp.int32, (NP,), 0)

    off = jnp.zeros((NE,), _f32)
    imp = jnp.zeros((NE,), _f32)
    for b in range(B):
        sb = scl_ref[b][:N][:, None]
        tb = t_ref[b] + jnp.where(sb > 0.0, g_ref[b, :N, :] * sb, 0.0)
        u1 = _ln(tb, ln1g_ref[...], ln1b_ref[...])
        qkv = jnp.dot(u1, wqkv_ref[...]) + bqkv_ref[...]
        outs = []
        for h in range(NH):
            q = qkv[:, h * DH:(h + 1) * DH]
            k = qkv[:, D + h * DH:D + (h + 1) * DH]
            v = qkv[:, 2 * D + h * DH:2 * D + (h + 1) * DH]
            s = jnp.dot(q, k.T) * (1.0 / math.sqrt(DH))
            p = jax.nn.softmax(s, axis=-1)
            outs.append(jnp.dot(p, v))
        o = jnp.concatenate(outs, axis=1)
        tm = tb + jnp.dot(o, wproj_ref[...]) + bproj_ref[...]
        tmid_ref[b] = tm
        u2 = _ln(tm, ln2g_ref[...], ln2b_ref[...])
        u_ref[b, :N, :] = u2
        u_ref[b, N:, :] = jnp.zeros((NP - N, D), _f32)

        logits = jnp.dot(u2, wg_ref[...])
        probs = jax.nn.softmax(logits, axis=-1)
        gate = jnp.max(probs, axis=-1)
        eq = probs == gate[:, None]
        idxf = jnp.min(jnp.where(eq, iota_e, 1e9), axis=-1)
        oh = (iota_e == idxf[:, None]).astype(_f32)
        cnt = jnp.dot(tril, oh)
        pos = jnp.sum((cnt + off[None, :]) * oh, axis=-1)
        keep = (pos < float(CAP)).astype(_f32)
        posc = jnp.minimum(pos, float(CAPP - 1))
        locb = (idxf * CAPP + posc).astype(jnp.int32)
        sclb = gate * keep
        loc_full = jnp.concatenate(
            [locb, jnp.full((NP - N,), CAPP - 1, jnp.int32)])
        scl_full = jnp.concatenate([sclb, jnp.zeros((NP - N,), _f32)])
        loc_ref[b] = jnp.where(lane < N, loc_full, CAPP - 1)
        sclo_ref[b] = jnp.where(lane < N, scl_full, 0.0)
        off = off + jnp.sum(oh, axis=0)
        imp = imp + jnp.sum(probs, axis=0)

    tot = float(B * N)
    aux = float(NE) * jnp.sum((imp / tot) * (off / tot))
    aux_ref[...] = aux.reshape(1, 1)


def _layer_head(t, g, scl, ln1g, ln1b, wqkv, bqkv, wproj, bproj, ln2g, ln2b, wg):
    return pl.pallas_call(
        _layer_body,
        out_shape=(
            jax.ShapeDtypeStruct((B, N, D), _f32),
            jax.ShapeDtypeStruct((B, NP, D), _f32),
            jax.ShapeDtypeStruct((B, NP), jnp.int32),
            jax.ShapeDtypeStruct((B, NP), _f32),
            jax.ShapeDtypeStruct((1, 1), _f32),
        ),
    )(t, g, scl, ln1g, ln1b, wqkv, bqkv, wproj, bproj, ln2g, ln2b, wg)


def _moe_body(u_ref, loc_ref, w1_ref, b1_ref, w2_ref, b2_ref, y_ref):
    e = pl.program_id(0)
    rowc = lax.broadcasted_iota(jnp.int32, (CAPP, NP), 0) + e * CAPP
    x = jnp.zeros((CAPP, D), _f32)
    disps = []
    for b in range(B):
        disp = (rowc == loc_ref[b][None, :]).astype(_f32)
        disps.append(disp)
        x = x + jnp.dot(disp, u_ref[b])
    h = jax.nn.gelu(jnp.dot(x, w1_ref[0]) + b1_ref[0])
    o = jnp.dot(h, w2_ref[0]) + b2_ref[0]
    for b in range(B):
        y = lax.dot_general(disps[b], o, (((0,), (0,)), ((), ())))

        @pl.when(e == 0)
        def _():
            y_ref[b] = y

        @pl.when(e > 0)
        def _():
            y_ref[b] += y


def _moe_ffn(u, loc2d, w1, b1, w2, b2):
    return pl.pallas_call(
        _moe_body,
        grid=(NE,),
        in_specs=[
            pl.BlockSpec((B, NP, D), lambda e: (0, 0, 0)),
            pl.BlockSpec((B, NP), lambda e: (0, 0)),
            pl.BlockSpec((1, D, HD), lambda e: (e, 0, 0)),
            pl.BlockSpec((1, 1, HD), lambda e: (e, 0, 0)),
            pl.BlockSpec((1, HD, D), lambda e: (e, 0, 0)),
            pl.BlockSpec((1, 1, D), lambda e: (e, 0, 0)),
        ],
        out_specs=pl.BlockSpec((B, NP, D), lambda e: (0, 0, 0)),
        out_shape=jax.ShapeDtypeStruct((B, NP, D), _f32),
    )(u, loc2d, w1, b1.reshape(NE, 1, HD), w2, b2.reshape(NE, 1, D))


def _head(t, g, scl, lnfg, lnfb, wh, bh, auxv):
    def body(t_ref, g_ref, scl_ref, lnfg_ref, lnfb_ref, wh_ref, bh_ref,
             aux_ref, logits_ref, cv_ref):
        rows = []
        for b in range(B):
            sb = scl_ref[b, 0]
            tb = t_ref[b, 0:1, :] + jnp.where(sb > 0.0,
                                              g_ref[b, 0:1, :] * sb, 0.0)
            rows.append(tb)
        tc = jnp.concatenate(rows, axis=0)
        tc = _ln(tc, lnfg_ref[...], lnfb_ref[...])
        logits_ref[...] = jnp.dot(tc, wh_ref[...]) + bh_ref[...]
        cv_ref[...] = jnp.sum(aux_ref[...]).reshape(1, 1)

    return pl.pallas_call(
        body,
        out_shape=(
            jax.ShapeDtypeStruct((B, 1000), _f32),
            jax.ShapeDtypeStruct((1, 1), _f32),
        ),
    )(t, g, scl, lnfg, lnfb, wh, bh, auxv)


@functools.lru_cache(maxsize=1)
def _sc_kernels():
    mesh = plsc.VectorSubcoreMesh(core_axis_name="c", subcore_axis_name="s")
    cparams = pltpu.CompilerParams(use_tc_tiling_on_sc=False)
    scratch = [
        pltpu.VMEM((PERW,), jnp.int32),
        pltpu.VMEM((PERW, D), _f32),
        pltpu.SemaphoreType.DMA,
    ]

    @functools.partial(
        pl.kernel,
        out_type=jax.ShapeDtypeStruct((NE * CAPP, D), _f32),
        mesh=mesh, scratch_types=scratch, compiler_params=cparams)
    def dispatch(u_hbm, loc_hbm, buf_hbm, loc_v, u_v, sem):
        wid = lax.axis_index("s") * 2 + lax.axis_index("c")
        base = wid * PERW
        pltpu.sync_copy(loc_hbm.at[pl.ds(base, PERW)], loc_v)
        pltpu.sync_copy(u_hbm.at[pl.ds(base, PERW)], u_v)
        pltpu.async_copy(u_v, buf_hbm.at[loc_v], sem).wait()

    @functools.partial(
        pl.kernel,
        out_type=jax.ShapeDtypeStruct((NTOK, D), _f32),
        mesh=mesh, scratch_types=scratch, compiler_params=cparams)
    def combine(outbuf_hbm, loc_hbm, g_hbm, loc_v, rows_v, sem):
        wid = lax.axis_index("s") * 2 + lax.axis_index("c")
        base = wid * PERW
        pltpu.sync_copy(loc_hbm.at[pl.ds(base, PERW)], loc_v)
        pltpu.async_copy(outbuf_hbm.at[loc_v], rows_v, sem).wait()
        pltpu.sync_copy(rows_v, g_hbm.at[pl.ds(base, PERW)])

    return dispatch, combine


def kernel(x, params):
    p = params
    gh = 224 // 16
    xp = (x.reshape(B, 3, gh, 16, gh, 16)
           .transpose(0, 2, 4, 1, 3, 5)
           .reshape(B * gh * gh, 3 * 16 * 16))
    t = _embed(xp, p['Wpatch'], p['bpatch'], p['cls'].reshape(1, D),
               p['pos'].reshape(N, D))

    g = jnp.zeros((B, NP, D), _f32)
    scl = jnp.zeros((B, NP), _f32)
    auxs = []
    for l in range(L):
        t, u, loc2d, scl2d, aux_l = _layer_head(
            t, g, scl, p['ln1_g'][l], p['ln1_b'][l], p['Wqkv'][l],
            p['bqkv'][l], p['Wproj'][l], p['bproj'][l], p['ln2_g'][l],
            p['ln2_b'][l], p['Wg'][l])
        g = _moe_ffn(u, loc2d, p['W1'][l], p['b1'][l], p['W2'][l], p['b2'][l])
        scl = scl2d
        auxs.append(aux_l.reshape(1))
    auxv = jnp.concatenate(auxs + [jnp.zeros((2,), _f32)]).reshape(1, 8)
    logits, cv = _head(t, g, scl, p['lnf_g'], p['lnf_b'], p['Whead'],
                       p['bhead'], auxv)
    return logits, cv.reshape(())


# speedup vs baseline: 1.1731x; 1.0080x over previous
import functools
import math

import jax
import jax.numpy as jnp
from jax import lax
from jax.experimental import pallas as pl
from jax.experimental.pallas import tpu as pltpu
from jax.experimental.pallas import tpu_sc as plsc

B = 8
N = 197
D = 192
NH = 3
DH = 64
L = 6
NE = 16
HD = 768
CAP = 197
CAPP = 256
NP = 256
NTOK = B * NP
NW = 32
PERW = NTOK // NW

_f32 = jnp.float32
_bf16 = jnp.bfloat16


def _bdot(a, b):
    return jax.lax.dot_general(
        a.astype(_bf16), b.astype(_bf16), (((a.ndim - 1,), (0,)), ((), ())),
        preferred_element_type=_f32)


def _ln(x, g, b):
    m = x.mean(-1, keepdims=True)
    v = ((x - m) ** 2).mean(-1, keepdims=True)
    return (x - m) / jnp.sqrt(v + 1e-6) * g + b


def _embed_body(xp_ref, wp_ref, bp_ref, cls_ref, pos_ref, out_ref):
    y = jnp.dot(xp_ref[...], wp_ref[...]) + bp_ref[...]
    for b in range(B):
        out_ref[b, 0:1, :] = cls_ref[...] + pos_ref[0:1, :]
        out_ref[b, 1:N, :] = y[b * (N - 1):(b + 1) * (N - 1), :] + pos_ref[1:N, :]


def _embed(xp, wp, bp, cls, pos):
    return pl.pallas_call(
        _embed_body,
        out_shape=jax.ShapeDtypeStruct((B, N, D), _f32),
    )(xp, wp, bp, cls, pos)


def _layer_body(t_ref, g_ref, scl_ref, ln1g_ref, ln1b_ref, wqkv_ref, bqkv_ref,
                wproj_ref, bproj_ref, ln2g_ref, ln2b_ref, wg_ref,
                tmid_ref, u_ref, loc_ref, sclo_ref, aux_ref):
    iota_e = lax.broadcasted_iota(jnp.int32, (N, NE), 1).astype(_f32)
    ii = lax.broadcasted_iota(jnp.int32, (N, N), 0)
    jj = lax.broadcasted_iota(jnp.int32, (N, N), 1)
    tril = (jj < ii).astype(_f32)
    lane = lax.broadcasted_iota(jnp.int32, (NP,), 0)

    off = jnp.zeros((NE,), _f32)
    imp = jnp.zeros((NE,), _f32)
    for b in range(B):
        sb = scl_ref[b][:N][:, None]
        tb = t_ref[b] + jnp.where(sb > 0.0, g_ref[b, :N, :] * sb, 0.0)
        u1 = _ln(tb, ln1g_ref[...], ln1b_ref[...])
        qkv = jnp.dot(u1, wqkv_ref[...]) + bqkv_ref[...]
        outs = []
        for h in range(NH):
            q = qkv[:, h * DH:(h + 1) * DH]
            k = qkv[:, D + h * DH:D + (h + 1) * DH]
            v = qkv[:, 2 * D + h * DH:2 * D + (h + 1) * DH]
            s = jnp.dot(q, k.T) * (1.0 / math.sqrt(DH))
            p = jax.nn.softmax(s, axis=-1)
            outs.append(jnp.dot(p, v))
        o = jnp.concatenate(outs, axis=1)
        tm = tb + jnp.dot(o, wproj_ref[...]) + bproj_ref[...]
        tmid_ref[b] = tm
        u2 = _ln(tm, ln2g_ref[...], ln2b_ref[...])
        u_ref[b, :N, :] = u2
        u_ref[b, N:, :] = jnp.zeros((NP - N, D), _f32)

        logits = jnp.dot(u2, wg_ref[...])
        probs = jax.nn.softmax(logits, axis=-1)
        gate = jnp.max(probs, axis=-1)
        eq = probs == gate[:, None]
        idxf = jnp.min(jnp.where(eq, iota_e, 1e9), axis=-1)
        oh = (iota_e == idxf[:, None]).astype(_f32)
        cnt = jnp.dot(tril, oh)
        pos = jnp.sum((cnt + off[None, :]) * oh, axis=-1)
        keep = (pos < float(CAP)).astype(_f32)
        posc = jnp.minimum(pos, float(CAPP - 1))
        locb = (idxf * CAPP + posc).astype(jnp.int32)
        sclb = gate * keep
        loc_full = jnp.concatenate(
            [locb, jnp.full((NP - N,), CAPP - 1, jnp.int32)])
        scl_full = jnp.concatenate([sclb, jnp.zeros((NP - N,), _f32)])
        loc_ref[b] = jnp.where(lane < N, loc_full, CAPP - 1)
        sclo_ref[b] = jnp.where(lane < N, scl_full, 0.0)
        off = off + jnp.sum(oh, axis=0)
        imp = imp + jnp.sum(probs, axis=0)

    tot = float(B * N)
    aux = float(NE) * jnp.sum((imp / tot) * (off / tot))
    aux_ref[...] = aux.reshape(1, 1)


def _layer_head(t, g, scl, ln1g, ln1b, wqkv, bqkv, wproj, bproj, ln2g, ln2b, wg):
    return pl.pallas_call(
        _layer_body,
        out_shape=(
            jax.ShapeDtypeStruct((B, N, D), _f32),
            jax.ShapeDtypeStruct((B, NP, D), _f32),
            jax.ShapeDtypeStruct((B, NP), jnp.int32),
            jax.ShapeDtypeStruct((B, NP), _f32),
            jax.ShapeDtypeStruct((1, 1), _f32),
        ),
    )(t, g, scl, ln1g, ln1b, wqkv, bqkv, wproj, bproj, ln2g, ln2b, wg)


def _moe_body(u_ref, loc_ref, w1_ref, b1_ref, w2_ref, b2_ref, y_ref):
    e = pl.program_id(0)
    rowc = lax.broadcasted_iota(jnp.int32, (CAPP, NP), 0) + e * CAPP
    x = jnp.zeros((CAPP, D), _f32)
    disps = []
    for b in range(B):
        disp = (rowc == loc_ref[b][None, :]).astype(_bf16)
        disps.append(disp)
        x = x + _bdot(disp, u_ref[b])
    h = jax.nn.gelu(_bdot(x, w1_ref[0]) + b1_ref[0])
    o = _bdot(h, w2_ref[0]) + b2_ref[0]
    for b in range(B):
        y = lax.dot_general(disps[b], o.astype(_bf16), (((0,), (0,)), ((), ())),
                            preferred_element_type=_f32)

        @pl.when(e == 0)
        def _():
            y_ref[b] = y

        @pl.when(e > 0)
        def _():
            y_ref[b] += y


def _moe_ffn(u, loc2d, w1, b1, w2, b2):
    return pl.pallas_call(
        _moe_body,
        grid=(NE,),
        in_specs=[
            pl.BlockSpec((B, NP, D), lambda e: (0, 0, 0)),
            pl.BlockSpec((B, NP), lambda e: (0, 0)),
            pl.BlockSpec((1, D, HD), lambda e: (e, 0, 0)),
            pl.BlockSpec((1, 1, HD), lambda e: (e, 0, 0)),
            pl.BlockSpec((1, HD, D), lambda e: (e, 0, 0)),
            pl.BlockSpec((1, 1, D), lambda e: (e, 0, 0)),
        ],
        out_specs=pl.BlockSpec((B, NP, D), lambda e: (0, 0, 0)),
        out_shape=jax.ShapeDtypeStruct((B, NP, D), _f32),
    )(u, loc2d, w1, b1.reshape(NE, 1, HD), w2, b2.reshape(NE, 1, D))


def _head(t, g, scl, lnfg, lnfb, wh, bh, auxv):
    def body(t_ref, g_ref, scl_ref, lnfg_ref, lnfb_ref, wh_ref, bh_ref,
             aux_ref, logits_ref, cv_ref):
        rows = []
        for b in range(B):
            sb = scl_ref[b, 0]
            tb = t_ref[b, 0:1, :] + jnp.where(sb > 0.0,
                                              g_ref[b, 0:1, :] * sb, 0.0)
            rows.append(tb)
        tc = jnp.concatenate(rows, axis=0)
        tc = _ln(tc, lnfg_ref[...], lnfb_ref[...])
        logits_ref[...] = jnp.dot(tc, wh_ref[...]) + bh_ref[...]
        cv_ref[...] = jnp.sum(aux_ref[...]).reshape(1, 1)

    return pl.pallas_call(
        body,
        out_shape=(
            jax.ShapeDtypeStruct((B, 1000), _f32),
            jax.ShapeDtypeStruct((1, 1), _f32),
        ),
    )(t, g, scl, lnfg, lnfb, wh, bh, auxv)


@functools.lru_cache(maxsize=1)
def _sc_kernels():
    mesh = plsc.VectorSubcoreMesh(core_axis_name="c", subcore_axis_name="s")
    cparams = pltpu.CompilerParams(use_tc_tiling_on_sc=False)
    scratch = [
        pltpu.VMEM((PERW,), jnp.int32),
        pltpu.VMEM((PERW, D), _f32),
        pltpu.SemaphoreType.DMA,
    ]

    @functools.partial(
        pl.kernel,
        out_type=jax.ShapeDtypeStruct((NE * CAPP, D), _f32),
        mesh=mesh, scratch_types=scratch, compiler_params=cparams)
    def dispatch(u_hbm, loc_hbm, buf_hbm, loc_v, u_v, sem):
        wid = lax.axis_index("s") * 2 + lax.axis_index("c")
        base = wid * PERW
        pltpu.sync_copy(loc_hbm.at[pl.ds(base, PERW)], loc_v)
        pltpu.sync_copy(u_hbm.at[pl.ds(base, PERW)], u_v)
        pltpu.async_copy(u_v, buf_hbm.at[loc_v], sem).wait()

    @functools.partial(
        pl.kernel,
        out_type=jax.ShapeDtypeStruct((NTOK, D), _f32),
        mesh=mesh, scratch_types=scratch, compiler_params=cparams)
    def combine(outbuf_hbm, loc_hbm, g_hbm, loc_v, rows_v, sem):
        wid = lax.axis_index("s") * 2 + lax.axis_index("c")
        base = wid * PERW
        pltpu.sync_copy(loc_hbm.at[pl.ds(base, PERW)], loc_v)
        pltpu.async_copy(outbuf_hbm.at[loc_v], rows_v, sem).wait()
        pltpu.sync_copy(rows_v, g_hbm.at[pl.ds(base, PERW)])

    return dispatch, combine


def kernel(x, params):
    p = params
    gh = 224 // 16
    xp = (x.reshape(B, 3, gh, 16, gh, 16)
           .transpose(0, 2, 4, 1, 3, 5)
           .reshape(B * gh * gh, 3 * 16 * 16))
    t = _embed(xp, p['Wpatch'], p['bpatch'], p['cls'].reshape(1, D),
               p['pos'].reshape(N, D))

    g = jnp.zeros((B, NP, D), _f32)
    scl = jnp.zeros((B, NP), _f32)
    auxs = []
    for l in range(L):
        t, u, loc2d, scl2d, aux_l = _layer_head(
            t, g, scl, p['ln1_g'][l], p['ln1_b'][l], p['Wqkv'][l],
            p['bqkv'][l], p['Wproj'][l], p['bproj'][l], p['ln2_g'][l],
            p['ln2_b'][l], p['Wg'][l])
        g = _moe_ffn(u, loc2d, p['W1'][l], p['b1'][l], p['W2'][l], p['b2'][l])
        scl = scl2d
        auxs.append(aux_l.reshape(1))
    auxv = jnp.concatenate(auxs + [jnp.zeros((2,), _f32)]).reshape(1, 8)
    logits, cv = _head(t, g, scl, p['lnf_g'], p['lnf_b'], p['Whead'],
                       p['bhead'], auxv)
    return logits, cv.reshape(())


# speedup vs baseline: 1.4551x; 1.2404x over previous
import math

import jax
import jax.numpy as jnp
from jax import lax
from jax.experimental import pallas as pl
from jax.experimental.pallas import tpu as pltpu

B = 8
N = 197
D = 192
NH = 3
DH = 64
L = 6
NE = 16
HD = 768
CAP = 197
CAPP = 256
NP = 256

_f32 = jnp.float32
_bf16 = jnp.bfloat16


def _bdot(a, b):
    return lax.dot_general(
        a.astype(_bf16), b.astype(_bf16), (((a.ndim - 1,), (0,)), ((), ())),
        preferred_element_type=_f32)


def _ln(x, g, b):
    m = x.mean(-1, keepdims=True)
    v = ((x - m) ** 2).mean(-1, keepdims=True)
    return (x - m) / jnp.sqrt(v + 1e-6) * g + b


def _embed_body(xp_ref, wp_ref, bp_ref, cls_ref, pos_ref, out_ref):
    y = jnp.dot(xp_ref[...], wp_ref[...]) + bp_ref[...]
    for b in range(B):
        out_ref[b, 0:1, :] = cls_ref[...] + pos_ref[0:1, :]
        out_ref[b, 1:N, :] = y[b * (N - 1):(b + 1) * (N - 1), :] + pos_ref[1:N, :]


def _embed(xp, wp, bp, cls, pos):
    return pl.pallas_call(
        _embed_body,
        out_shape=jax.ShapeDtypeStruct((B, N, D), _f32),
    )(xp, wp, bp, cls, pos)


def _encoder_body(t0_ref, ln1g_ref, ln1b_ref, wqkv_ref, bqkv_ref, wproj_ref,
                  bproj_ref, ln2g_ref, ln2b_ref, wg_ref, w1_ref, b1_ref,
                  w2_ref, b2_ref, lnfg_ref, lnfb_ref, wh_ref, bh_ref,
                  logits_ref, cv_ref,
                  t_s, u_s, y_s, loc_s, scl_s, aux_s):
    l = pl.program_id(0)
    e = pl.program_id(1)

    @pl.when(e == 0)
    def _prologue():
        iota_e = lax.broadcasted_iota(jnp.int32, (N, NE), 1).astype(_f32)
        ii = lax.broadcasted_iota(jnp.int32, (N, N), 0)
        jj = lax.broadcasted_iota(jnp.int32, (N, N), 1)
        tril = (jj < ii).astype(_f32)
        lane = lax.broadcasted_iota(jnp.int32, (NP,), 0)
        is_l0 = l == 0

        off = jnp.zeros((NE,), _f32)
        imp = jnp.zeros((NE,), _f32)
        for b in range(B):
            sb = scl_s[b][:N][:, None]
            fold = t_s[b, :N, :] + jnp.where(sb > 0.0,
                                             y_s[b, :N, :] * sb, 0.0)
            tb = jnp.where(is_l0, t0_ref[b], fold)
            u1 = _ln(tb, ln1g_ref[0], ln1b_ref[0])
            qkv = jnp.dot(u1, wqkv_ref[0]) + bqkv_ref[0]
            outs = []
            for h in range(NH):
                q = qkv[:, h * DH:(h + 1) * DH]
                k = qkv[:, D + h * DH:D + (h + 1) * DH]
                v = qkv[:, 2 * D + h * DH:2 * D + (h + 1) * DH]
                s = jnp.dot(q, k.T) * (1.0 / math.sqrt(DH))
                p = jax.nn.softmax(s, axis=-1)
                outs.append(jnp.dot(p, v))
            o = jnp.concatenate(outs, axis=1)
            tm = tb + jnp.dot(o, wproj_ref[0]) + bproj_ref[0]
            t_s[b, :N, :] = tm
            u2 = _ln(tm, ln2g_ref[0], ln2b_ref[0])
            u_s[b, :N, :] = u2
            u_s[b, N:, :] = jnp.zeros((NP - N, D), _f32)

            logits = jnp.dot(u2, wg_ref[0])
            probs = jax.nn.softmax(logits, axis=-1)
            gate = jnp.max(probs, axis=-1)
            eq = probs == gate[:, None]
            idxf = jnp.min(jnp.where(eq, iota_e, 1e9), axis=-1)
            oh = (iota_e == idxf[:, None]).astype(_f32)
            cnt = jnp.dot(tril, oh)
            pos = jnp.sum((cnt + off[None, :]) * oh, axis=-1)
            keep = (pos < float(CAP)).astype(_f32)
            posc = jnp.minimum(pos, float(CAPP - 1))
            locb = (idxf * CAPP + posc).astype(jnp.int32)
            sclb = gate * keep
            loc_full = jnp.concatenate(
                [locb, jnp.full((NP - N,), CAPP - 1, jnp.int32)])
            scl_full = jnp.concatenate([sclb, jnp.zeros((NP - N,), _f32)])
            loc_s[b] = jnp.where(lane < N, loc_full, CAPP - 1)
            scl_s[b] = jnp.where(lane < N, scl_full, 0.0)
            off = off + jnp.sum(oh, axis=0)
            imp = imp + jnp.sum(probs, axis=0)

        tot = float(B * N)
        aux = float(NE) * jnp.sum((imp / tot) * (off / tot))
        prev = aux_s[0, 0]
        aux_s[...] = jnp.where(is_l0, aux, prev + aux).reshape(1, 1)

    rowc = lax.broadcasted_iota(jnp.int32, (CAPP, NP), 0) + e * CAPP
    disps = []
    x = jnp.zeros((CAPP, D), _f32)
    for b in range(B):
        disp = (rowc == loc_s[b][None, :]).astype(_bf16)
        disps.append(disp)
        x = x + _bdot(disp, u_s[b])
    h = jax.nn.gelu(_bdot(x, w1_ref[0, 0]) + b1_ref[0, 0])
    o = _bdot(h, w2_ref[0, 0]) + b2_ref[0, 0]
    ob = o.astype(_bf16)
    for b in range(B):
        yb = lax.dot_general(disps[b], ob, (((0,), (0,)), ((), ())),
                             preferred_element_type=_f32)

        @pl.when(e == 0)
        def _():
            y_s[b] = yb

        @pl.when(e > 0)
        def _():
            y_s[b] += yb

    @pl.when(jnp.logical_and(l == L - 1, e == NE - 1))
    def _head():
        rows = []
        for b in range(B):
            sb = scl_s[b, 0]
            tb = t_s[b, 0:1, :] + jnp.where(sb > 0.0,
                                            y_s[b, 0:1, :] * sb, 0.0)
            rows.append(tb)
        tc = _ln(jnp.concatenate(rows, axis=0), lnfg_ref[0], lnfb_ref[0])
        logits_ref[...] = jnp.dot(tc, wh_ref[...]) + bh_ref[...]
        cv_ref[...] = aux_s[...]


def _encoder(t0, p):
    specs = [
        pl.BlockSpec((B, N, D), lambda l, e: (0, 0, 0)),
        pl.BlockSpec((1, 1, D), lambda l, e: (l, 0, 0)),
        pl.BlockSpec((1, 1, D), lambda l, e: (l, 0, 0)),
        pl.BlockSpec((1, D, 3 * D), lambda l, e: (l, 0, 0)),
        pl.BlockSpec((1, 1, 3 * D), lambda l, e: (l, 0, 0)),
        pl.BlockSpec((1, D, D), lambda l, e: (l, 0, 0)),
        pl.BlockSpec((1, 1, D), lambda l, e: (l, 0, 0)),
        pl.BlockSpec((1, 1, D), lambda l, e: (l, 0, 0)),
        pl.BlockSpec((1, 1, D), lambda l, e: (l, 0, 0)),
        pl.BlockSpec((1, D, NE), lambda l, e: (l, 0, 0)),
        pl.BlockSpec((1, 1, D, HD), lambda l, e: (l, e, 0, 0)),
        pl.BlockSpec((1, 1, 1, HD), lambda l, e: (l, e, 0, 0)),
        pl.BlockSpec((1, 1, HD, D), lambda l, e: (l, e, 0, 0)),
        pl.BlockSpec((1, 1, 1, D), lambda l, e: (l, e, 0, 0)),
        pl.BlockSpec((1, D), lambda l, e: (0, 0)),
        pl.BlockSpec((1, D), lambda l, e: (0, 0)),
        pl.BlockSpec((D, 1000), lambda l, e: (0, 0)),
        pl.BlockSpec((1, 1000), lambda l, e: (0, 0)),
    ]
    return pl.pallas_call(
        _encoder_body,
        grid=(L, NE),
        in_specs=specs,
        out_specs=(
            pl.BlockSpec((B, 1000), lambda l, e: (0, 0)),
            pl.BlockSpec((1, 1), lambda l, e: (0, 0)),
        ),
        out_shape=(
            jax.ShapeDtypeStruct((B, 1000), _f32),
            jax.ShapeDtypeStruct((1, 1), _f32),
        ),
        scratch_shapes=[
            pltpu.VMEM((B, NP, D), _f32),
            pltpu.VMEM((B, NP, D), _f32),
            pltpu.VMEM((B, NP, D), _f32),
            pltpu.VMEM((B, NP), jnp.int32),
            pltpu.VMEM((B, NP), _f32),
            pltpu.VMEM((1, 1), _f32),
        ],
    )(
        t0,
        p['ln1_g'].reshape(L, 1, D), p['ln1_b'].reshape(L, 1, D),
        p['Wqkv'], p['bqkv'].reshape(L, 1, 3 * D),
        p['Wproj'], p['bproj'].reshape(L, 1, D),
        p['ln2_g'].reshape(L, 1, D), p['ln2_b'].reshape(L, 1, D),
        p['Wg'],
        p['W1'], p['b1'].reshape(L, NE, 1, HD),
        p['W2'], p['b2'].reshape(L, NE, 1, D),
        p['lnf_g'].reshape(1, D), p['lnf_b'].reshape(1, D),
        p['Whead'], p['bhead'].reshape(1, 1000),
    )


def kernel(x, params):
    p = params
    gh = 224 // 16
    xp = (x.reshape(B, 3, gh, 16, gh, 16)
           .transpose(0, 2, 4, 1, 3, 5)
           .reshape(B * gh * gh, 3 * 16 * 16))
    t0 = _embed(xp, p['Wpatch'], p['bpatch'], p['cls'].reshape(1, D),
                p['pos'].reshape(N, D))
    logits, cv = _encoder(t0, p)
    return logits, cv.reshape(())


# speedup vs baseline: 1.5246x; 1.0478x over previous
import math

import jax
import jax.numpy as jnp
from jax import lax
from jax.experimental import pallas as pl
from jax.experimental.pallas import tpu as pltpu

B = 8
N = 197
D = 192
NH = 3
DH = 64
L = 6
NE = 16
HD = 768
CAP = 197
CAPP = 208
NP = 256

_f32 = jnp.float32
_bf16 = jnp.bfloat16


def _bdot(a, b):
    return lax.dot_general(
        a.astype(_bf16), b.astype(_bf16), (((a.ndim - 1,), (0,)), ((), ())),
        preferred_element_type=_f32)


def _ln(x, g, b):
    m = x.mean(-1, keepdims=True)
    v = ((x - m) ** 2).mean(-1, keepdims=True)
    return (x - m) * lax.rsqrt(v + 1e-6) * g + b


def _softmax(s):
    p = jnp.exp(s)
    return p * (1.0 / jnp.sum(p, axis=-1, keepdims=True))


def _embed_body(xp_ref, wp_ref, bp_ref, cls_ref, pos_ref, out_ref):
    y = jnp.dot(xp_ref[...], wp_ref[...]) + bp_ref[...]
    for b in range(B):
        out_ref[b, 0:1, :] = cls_ref[...] + pos_ref[0:1, :]
        out_ref[b, 1:N, :] = y[b * (N - 1):(b + 1) * (N - 1), :] + pos_ref[1:N, :]


def _embed(xp, wp, bp, cls, pos):
    return pl.pallas_call(
        _embed_body,
        out_shape=jax.ShapeDtypeStruct((B, N, D), _f32),
    )(xp, wp, bp, cls, pos)


def _encoder_body(t0_ref, ln1g_ref, ln1b_ref, wqkv_ref, bqkv_ref, wproj_ref,
                  bproj_ref, ln2g_ref, ln2b_ref, wg_ref, w1_ref, b1_ref,
                  w2_ref, b2_ref, lnfg_ref, lnfb_ref, wh_ref, bh_ref,
                  logits_ref, cv_ref,
                  t_s, u_s, y_s, a_s, idx_s, scl_s, aux_s):
    l = pl.program_id(0)
    e = pl.program_id(1)

    @pl.when(jnp.logical_and(l == 0, e == 0))
    def _zero_pad():
        for b in range(B):
            u_s[b, N:, :] = jnp.zeros((NP - N, D), _bf16)

    @pl.when(e == 0)
    def _prologue():
        iota_e = lax.broadcasted_iota(jnp.int32, (N, NE), 1).astype(_f32)
        ii = lax.broadcasted_iota(jnp.int32, (N, N), 0)
        jj = lax.broadcasted_iota(jnp.int32, (N, N), 1)
        tril = (jj < ii).astype(_bf16)
        lane = lax.broadcasted_iota(jnp.int32, (NP,), 0)
        iota_c = lax.broadcasted_iota(jnp.int32, (NP, CAPP), 1)
        is_l0 = l == 0

        off = jnp.zeros((NE,), _f32)
        imp = jnp.zeros((NE,), _f32)
        for b in range(B):
            sb = scl_s[b][:N][:, None]
            fold = t_s[b, :N, :] + jnp.where(sb > 0.0,
                                             y_s[b, :N, :] * sb, 0.0)
            tb = jnp.where(is_l0, t0_ref[b], fold)
            u1 = _ln(tb, ln1g_ref[0], ln1b_ref[0])
            qkv = _bdot(u1, wqkv_ref[0]) + bqkv_ref[0]
            outs = []
            for h in range(NH):
                q = qkv[:, h * DH:(h + 1) * DH]
                k = qkv[:, D + h * DH:D + (h + 1) * DH]
                v = qkv[:, 2 * D + h * DH:2 * D + (h + 1) * DH]
                s = lax.dot_general(
                    q.astype(_bf16), k.astype(_bf16),
                    (((1,), (1,)), ((), ())),
                    preferred_element_type=_f32) * (1.0 / math.sqrt(DH))
                p = _softmax(s)
                outs.append(_bdot(p, v))
            o = jnp.concatenate(outs, axis=1)
            tm = tb + _bdot(o, wproj_ref[0]) + bproj_ref[0]
            t_s[b, :N, :] = tm
            u2 = _ln(tm, ln2g_ref[0], ln2b_ref[0])
            u_s[b, :N, :] = u2.astype(_bf16)

            logits = jnp.dot(u2, wg_ref[0])
            probs = _softmax(logits)
            gate = jnp.max(probs, axis=-1)
            eq = probs == gate[:, None]
            idxf = jnp.min(jnp.where(eq, iota_e, 1e9), axis=-1)
            oh = (iota_e == idxf[:, None]).astype(_f32)
            cnt = _bdot(tril, oh)
            pos = jnp.sum((cnt + off[None, :]) * oh, axis=-1)
            keep = (pos < float(CAP)).astype(_f32)
            posc = jnp.minimum(pos, float(CAPP - 1)).astype(jnp.int32)
            sclb = gate * keep
            idx_full = jnp.concatenate(
                [idxf.astype(jnp.int32), jnp.full((NP - N,), NE, jnp.int32)])
            pos_full = jnp.concatenate(
                [posc, jnp.full((NP - N,), CAPP - 1, jnp.int32)])
            scl_full = jnp.concatenate([sclb, jnp.zeros((NP - N,), _f32)])
            idx_s[b] = jnp.where(lane < N, idx_full, NE)
            scl_s[b] = jnp.where(lane < N, scl_full, 0.0)
            a_s[b] = (pos_full[:, None] == iota_c).astype(_bf16)
            off = off + jnp.sum(oh, axis=0)
            imp = imp + jnp.sum(probs, axis=0)

        tot = float(B * N)
        aux = float(NE) * jnp.sum((imp / tot) * (off / tot))
        prev = aux_s[0, 0]
        aux_s[...] = jnp.where(is_l0, aux, prev + aux).reshape(1, 1)

    x = jnp.zeros((CAPP, D), _f32)
    masks = []
    for b in range(B):
        m = (idx_s[b] == e).astype(_bf16)[:, None]
        masks.append(m)
        x = x + lax.dot_general(a_s[b], u_s[b] * m, (((0,), (0,)), ((), ())),
                                preferred_element_type=_f32)
    h = jax.nn.gelu(_bdot(x, w1_ref[0, 0]) + b1_ref[0, 0])
    o = _bdot(h, w2_ref[0, 0]) + b2_ref[0, 0]
    ob = o.astype(_bf16)
    for b in range(B):
        yb = jnp.dot(a_s[b], ob, preferred_element_type=_f32)
        yb = yb * masks[b].astype(_f32)

        @pl.when(e == 0)
        def _():
            y_s[b] = yb

        @pl.when(e > 0)
        def _():
            y_s[b] += yb

    @pl.when(jnp.logical_and(l == L - 1, e == NE - 1))
    def _head():
        rows = []
        for b in range(B):
            sb = scl_s[b, 0]
            tb = t_s[b, 0:1, :] + jnp.where(sb > 0.0,
                                            y_s[b, 0:1, :] * sb, 0.0)
            rows.append(tb)
        tc = _ln(jnp.concatenate(rows, axis=0), lnfg_ref[0], lnfb_ref[0])
        logits_ref[...] = jnp.dot(tc, wh_ref[...]) + bh_ref[...]
        cv_ref[...] = aux_s[...]


def _encoder(t0, p):
    specs = [
        pl.BlockSpec((B, N, D), lambda l, e: (0, 0, 0)),
        pl.BlockSpec((1, 1, D), lambda l, e: (l, 0, 0)),
        pl.BlockSpec((1, 1, D), lambda l, e: (l, 0, 0)),
        pl.BlockSpec((1, D, 3 * D), lambda l, e: (l, 0, 0)),
        pl.BlockSpec((1, 1, 3 * D), lambda l, e: (l, 0, 0)),
        pl.BlockSpec((1, D, D), lambda l, e: (l, 0, 0)),
        pl.BlockSpec((1, 1, D), lambda l, e: (l, 0, 0)),
        pl.BlockSpec((1, 1, D), lambda l, e: (l, 0, 0)),
        pl.BlockSpec((1, 1, D), lambda l, e: (l, 0, 0)),
        pl.BlockSpec((1, D, NE), lambda l, e: (l, 0, 0)),
        pl.BlockSpec((1, 1, D, HD), lambda l, e: (l, e, 0, 0)),
        pl.BlockSpec((1, 1, 1, HD), lambda l, e: (l, e, 0, 0)),
        pl.BlockSpec((1, 1, HD, D), lambda l, e: (l, e, 0, 0)),
        pl.BlockSpec((1, 1, 1, D), lambda l, e: (l, e, 0, 0)),
        pl.BlockSpec((1, D), lambda l, e: (0, 0)),
        pl.BlockSpec((1, D), lambda l, e: (0, 0)),
        pl.BlockSpec((D, 1000), lambda l, e: (0, 0)),
        pl.BlockSpec((1, 1000), lambda l, e: (0, 0)),
    ]
    return pl.pallas_call(
        _encoder_body,
        grid=(L, NE),
        in_specs=specs,
        out_specs=(
            pl.BlockSpec((B, 1000), lambda l, e: (0, 0)),
            pl.BlockSpec((1, 1), lambda l, e: (0, 0)),
        ),
        out_shape=(
            jax.ShapeDtypeStruct((B, 1000), _f32),
            jax.ShapeDtypeStruct((1, 1), _f32),
        ),
        scratch_shapes=[
            pltpu.VMEM((B, NP, D), _f32),
            pltpu.VMEM((B, NP, D), _bf16),
            pltpu.VMEM((B, NP, D), _f32),
            pltpu.VMEM((B, NP, CAPP), _bf16),
            pltpu.VMEM((B, NP), jnp.int32),
            pltpu.VMEM((B, NP), _f32),
            pltpu.VMEM((1, 1), _f32),
        ],
    )(
        t0,
        p['ln1_g'].reshape(L, 1, D), p['ln1_b'].reshape(L, 1, D),
        p['Wqkv'], p['bqkv'].reshape(L, 1, 3 * D),
        p['Wproj'], p['bproj'].reshape(L, 1, D),
        p['ln2_g'].reshape(L, 1, D), p['ln2_b'].reshape(L, 1, D),
        p['Wg'],
        p['W1'], p['b1'].reshape(L, NE, 1, HD),
        p['W2'], p['b2'].reshape(L, NE, 1, D),
        p['lnf_g'].reshape(1, D), p['lnf_b'].reshape(1, D),
        p['Whead'], p['bhead'].reshape(1, 1000),
    )


def kernel(x, params):
    p = params
    gh = 224 // 16
    xp = (x.reshape(B, 3, gh, 16, gh, 16)
           .transpose(0, 2, 4, 1, 3, 5)
           .reshape(B * gh * gh, 3 * 16 * 16))
    t0 = _embed(xp, p['Wpatch'], p['bpatch'], p['cls'].reshape(1, D),
                p['pos'].reshape(N, D))
    logits, cv = _encoder(t0, p)
    return logits, cv.reshape(())


# speedup vs baseline: 1.9242x; 1.2621x over previous
import math

import jax
import jax.numpy as jnp
from jax import lax
from jax.experimental import pallas as pl
from jax.experimental.pallas import tpu as pltpu

B = 8
N = 197
D = 192
NH = 3
DH = 64
L = 6
NE = 16
HD = 768
CAP = 197
CAPP = 208
NP = 256

_f32 = jnp.float32
_bf16 = jnp.bfloat16


def _bdot(a, b):
    return lax.dot_general(
        a.astype(_bf16), b.astype(_bf16), (((a.ndim - 1,), (0,)), ((), ())),
        preferred_element_type=_f32)


def _ln(x, g, b):
    m = x.mean(-1, keepdims=True)
    v = ((x - m) ** 2).mean(-1, keepdims=True)
    return (x - m) * lax.rsqrt(v + 1e-6) * g + b


def _softmax(s):
    p = jnp.exp(s)
    return p * (1.0 / jnp.sum(p, axis=-1, keepdims=True))


def _embed_body(xp_ref, wp_ref, bp_ref, cls_ref, pos_ref, out_ref):
    y = jnp.dot(xp_ref[...], wp_ref[...]) + bp_ref[...]
    for b in range(B):
        out_ref[b, 0:1, :] = cls_ref[...] + pos_ref[0:1, :]
        out_ref[b, 1:N, :] = y[b * (N - 1):(b + 1) * (N - 1), :] + pos_ref[1:N, :]


def _embed(xp, wp, bp, cls, pos):
    return pl.pallas_call(
        _embed_body,
        out_shape=jax.ShapeDtypeStruct((B, N, D), _f32),
    )(xp, wp, bp, cls, pos)


def _encoder_body(t0_ref, ln1g_ref, ln1b_ref, wqkv_ref, bqkv_ref, wproj_ref,
                  bproj_ref, ln2g_ref, ln2b_ref, wg_ref, w1_ref, b1_ref,
                  w2_ref, b2_ref, lnfg_ref, lnfb_ref, wh_ref, bh_ref,
                  logits_ref, cv_ref,
                  t_s, u_s, y_s, a_s, idx_s, scl_s, aux_s):
    l = pl.program_id(0)
    e = pl.program_id(1)

    @pl.when(jnp.logical_and(l == 0, e == 0))
    def _zero_pad():
        for b in range(B):
            u_s[pl.ds(b * NP + N, NP - N), :] = jnp.zeros((NP - N, D), _bf16)

    @pl.when(e == 0)
    def _prologue():
        iota_e = lax.broadcasted_iota(jnp.int32, (N, NE), 1).astype(_f32)
        ii = lax.broadcasted_iota(jnp.int32, (N, N), 0)
        jj = lax.broadcasted_iota(jnp.int32, (N, N), 1)
        tril = (jj < ii).astype(_bf16)
        lane = lax.broadcasted_iota(jnp.int32, (NP,), 0)
        iota_c = lax.broadcasted_iota(jnp.int32, (NP, CAPP), 1)
        is_l0 = l == 0

        off = jnp.zeros((NE,), _f32)
        imp = jnp.zeros((NE,), _f32)
        for b in range(B):
            sb = scl_s[b][:N][:, None]
            yrows = y_s[pl.ds(b * NP, N), :].astype(_f32)
            fold = t_s[b, :N, :] + jnp.where(sb > 0.0, yrows * sb, 0.0)
            tb = jnp.where(is_l0, t0_ref[b], fold)
            u1 = _ln(tb, ln1g_ref[0], ln1b_ref[0])
            qkv = _bdot(u1, wqkv_ref[0]) + bqkv_ref[0]
            outs = []
            for h in range(NH):
                q = qkv[:, h * DH:(h + 1) * DH]
                k = qkv[:, D + h * DH:D + (h + 1) * DH]
                v = qkv[:, 2 * D + h * DH:2 * D + (h + 1) * DH]
                s = lax.dot_general(
                    q.astype(_bf16), k.astype(_bf16),
                    (((1,), (1,)), ((), ())),
                    preferred_element_type=_f32) * (1.0 / math.sqrt(DH))
                p = _softmax(s)
                outs.append(_bdot(p, v))
            o = jnp.concatenate(outs, axis=1)
            tm = tb + _bdot(o, wproj_ref[0]) + bproj_ref[0]
            t_s[b, :N, :] = tm
            u2 = _ln(tm, ln2g_ref[0], ln2b_ref[0])
            u_s[pl.ds(b * NP, N), :] = u2.astype(_bf16)

            logits = jnp.dot(u2, wg_ref[0])
            probs = _softmax(logits)
            gate = jnp.max(probs, axis=-1)
            eq = probs == gate[:, None]
            idxf = jnp.min(jnp.where(eq, iota_e, 1e9), axis=-1)
            oh = (iota_e == idxf[:, None]).astype(_f32)
            cnt = _bdot(tril, oh)
            pos = jnp.sum((cnt + off[None, :]) * oh, axis=-1)
            keep = (pos < float(CAP)).astype(_f32)
            posc = jnp.minimum(pos, float(CAPP - 1)).astype(jnp.int32)
            sclb = gate * keep
            idx_full = jnp.concatenate(
                [idxf.astype(jnp.int32), jnp.full((NP - N,), NE, jnp.int32)])
            pos_full = jnp.concatenate(
                [posc, jnp.full((NP - N,), CAPP - 1, jnp.int32)])
            scl_full = jnp.concatenate([sclb, jnp.zeros((NP - N,), _f32)])
            idx_s[pl.ds(b * NP, NP)] = jnp.where(lane < N, idx_full, NE)
            scl_s[b] = jnp.where(lane < N, scl_full, 0.0)
            a_s[pl.ds(b * NP, NP), :] = (pos_full[:, None] == iota_c).astype(_bf16)
            off = off + jnp.sum(oh, axis=0)
            imp = imp + jnp.sum(probs, axis=0)

        tot = float(B * N)
        aux = float(NE) * jnp.sum((imp / tot) * (off / tot))
        prev = aux_s[0, 0]
        aux_s[...] = jnp.where(is_l0, aux, prev + aux).reshape(1, 1)

    m = (idx_s[...] == e).astype(_bf16)[:, None]
    um = u_s[...] * m
    x = lax.dot_general(a_s[...], um, (((0,), (0,)), ((), ())),
                        preferred_element_type=_f32)
    h = jax.nn.gelu(_bdot(x, w1_ref[0, 0]) + b1_ref[0, 0])
    o = _bdot(h, w2_ref[0, 0]) + b2_ref[0, 0]
    yb = jnp.dot(a_s[...], o.astype(_bf16),
                 preferred_element_type=_f32)
    ym = (yb * m.astype(_f32)).astype(_bf16)

    @pl.when(e == 0)
    def _():
        y_s[...] = ym

    @pl.when(e > 0)
    def _():
        y_s[...] += ym

    @pl.when(jnp.logical_and(l == L - 1, e == NE - 1))
    def _head():
        rows = []
        for b in range(B):
            sb = scl_s[b, 0]
            ycls = y_s[pl.ds(b * NP, 1), :].astype(_f32)
            tb = t_s[b, 0:1, :] + jnp.where(sb > 0.0, ycls * sb, 0.0)
            rows.append(tb)
        tc = _ln(jnp.concatenate(rows, axis=0), lnfg_ref[0], lnfb_ref[0])
        logits_ref[...] = jnp.dot(tc, wh_ref[...]) + bh_ref[...]
        cv_ref[...] = aux_s[...]


def _encoder(t0, p):
    specs = [
        pl.BlockSpec((B, N, D), lambda l, e: (0, 0, 0)),
        pl.BlockSpec((1, 1, D), lambda l, e: (l, 0, 0)),
        pl.BlockSpec((1, 1, D), lambda l, e: (l, 0, 0)),
        pl.BlockSpec((1, D, 3 * D), lambda l, e: (l, 0, 0)),
        pl.BlockSpec((1, 1, 3 * D), lambda l, e: (l, 0, 0)),
        pl.BlockSpec((1, D, D), lambda l, e: (l, 0, 0)),
        pl.BlockSpec((1, 1, D), lambda l, e: (l, 0, 0)),
        pl.BlockSpec((1, 1, D), lambda l, e: (l, 0, 0)),
        pl.BlockSpec((1, 1, D), lambda l, e: (l, 0, 0)),
        pl.BlockSpec((1, D, NE), lambda l, e: (l, 0, 0)),
        pl.BlockSpec((1, 1, D, HD), lambda l, e: (l, e, 0, 0)),
        pl.BlockSpec((1, 1, 1, HD), lambda l, e: (l, e, 0, 0)),
        pl.BlockSpec((1, 1, HD, D), lambda l, e: (l, e, 0, 0)),
        pl.BlockSpec((1, 1, 1, D), lambda l, e: (l, e, 0, 0)),
        pl.BlockSpec((1, D), lambda l, e: (0, 0)),
        pl.BlockSpec((1, D), lambda l, e: (0, 0)),
        pl.BlockSpec((D, 1000), lambda l, e: (0, 0)),
        pl.BlockSpec((1, 1000), lambda l, e: (0, 0)),
    ]
    return pl.pallas_call(
        _encoder_body,
        grid=(L, NE),
        in_specs=specs,
        out_specs=(
            pl.BlockSpec((B, 1000), lambda l, e: (0, 0)),
            pl.BlockSpec((1, 1), lambda l, e: (0, 0)),
        ),
        out_shape=(
            jax.ShapeDtypeStruct((B, 1000), _f32),
            jax.ShapeDtypeStruct((1, 1), _f32),
        ),
        scratch_shapes=[
            pltpu.VMEM((B, NP, D), _f32),
            pltpu.VMEM((B * NP, D), _bf16),
            pltpu.VMEM((B * NP, D), _bf16),
            pltpu.VMEM((B * NP, CAPP), _bf16),
            pltpu.VMEM((B * NP,), jnp.int32),
            pltpu.VMEM((B, NP), _f32),
            pltpu.VMEM((1, 1), _f32),
        ],
    )(
        t0,
        p['ln1_g'].reshape(L, 1, D), p['ln1_b'].reshape(L, 1, D),
        p['Wqkv'], p['bqkv'].reshape(L, 1, 3 * D),
        p['Wproj'], p['bproj'].reshape(L, 1, D),
        p['ln2_g'].reshape(L, 1, D), p['ln2_b'].reshape(L, 1, D),
        p['Wg'],
        p['W1'], p['b1'].reshape(L, NE, 1, HD),
        p['W2'], p['b2'].reshape(L, NE, 1, D),
        p['lnf_g'].reshape(1, D), p['lnf_b'].reshape(1, D),
        p['Whead'], p['bhead'].reshape(1, 1000),
    )


def kernel(x, params):
    p = params
    gh = 224 // 16
    xp = (x.reshape(B, 3, gh, 16, gh, 16)
           .transpose(0, 2, 4, 1, 3, 5)
           .reshape(B * gh * gh, 3 * 16 * 16))
    t0 = _embed(xp, p['Wpatch'], p['bpatch'], p['cls'].reshape(1, D),
                p['pos'].reshape(N, D))
    logits, cv = _encoder(t0, p)
    return logits, cv.reshape(())
